# Initial kernel scaffold; baseline (speedup 1.0000x reference)
import jax
import jax.numpy as jnp
from jax.experimental import pallas as pl


def kernel(X, params):
    raise NotImplementedError("write your pallas kernel here")


# speedup vs baseline: 14.4177x; 14.4177x over previous
import functools

import jax
import jax.numpy as jnp
from jax import lax
from jax.experimental import pallas as pl
from jax.experimental.pallas import tpu as pltpu
from jax.experimental.pallas import tpu_sc as plsc

L = 16
NC = 2
NS = 16
NW = NC * NS

_CES = [(0, 1), (2, 3), (4, 5), (6, 7),
        (0, 2), (1, 3), (4, 6), (5, 7),
        (1, 2), (5, 6),
        (0, 4), (1, 5), (2, 6), (3, 7),
        (2, 4), (3, 5),
        (1, 2), (3, 4), (5, 6)]


def _make_kernel(B, I, A, T, D, BC):
    NF = I // NW
    NG = BC // L
    mesh = plsc.VectorSubcoreMesh(core_axis_name="c", subcore_axis_name="s",
                                  num_cores=NC, num_subcores=NS)

    @functools.partial(
        pl.kernel,
        out_type=jax.ShapeDtypeStruct((I * B * D,), jnp.float32),
        mesh=mesh,
        scratch_types=[
            pltpu.VMEM((NF * T * D,), jnp.float32),
            pltpu.VMEM((NF * A * B,), jnp.float32),
            pltpu.VMEM((NF * BC * D,), jnp.float32),
        ],
        compiler_params=pltpu.CompilerParams(needs_layout_passes=False),
    )
    def k(xt_hbm, pflat_hbm, yt_hbm, tb, xb, ob):
        wid = lax.axis_index("s") * NC + lax.axis_index("c")
        f0 = wid * NF
        pltpu.sync_copy(pflat_hbm.at[pl.ds(f0 * (T * D), NF * T * D)], tb)
        pltpu.sync_copy(xt_hbm.at[pl.ds(f0 * (A * B), NF * A * B)], xb)
        lane = jnp.arange(L, dtype=jnp.int32)
        laneD = lane * D
        pay = [jnp.full((L,), 1 << j, jnp.int32) for j in range(A)]

        for ci in range(B // BC):
            b0c = ci * BC

            def fbody(fl, carry):
                xfl = fl * (A * B)
                tbase = fl * (T * D)
                obase_f = fl * (BC * D)

                def gbody(g, carry2):
                    b0 = b0c + g * L
                    v = [xb[pl.ds(xfl + j * B + b0, L)] for j in range(A)]
                    p = list(pay)
                    for a, b in _CES:
                        c = v[a] <= v[b]
                        va, vb = v[a], v[b]
                        v[a] = jnp.where(c, va, vb)
                        v[b] = jnp.where(c, vb, va)
                        pa, pb = p[a], p[b]
                        p[a] = jnp.where(c, pa, pb)
                        p[b] = jnp.where(c, pb, pa)
                    coef = [v[0]] + [v[j] - v[j - 1] for j in range(1, A)]
                    m = [None] * A
                    m[A - 1] = p[A - 1]
                    for j in range(A - 2, 0, -1):
                        m[j] = m[j + 1] + p[j]
                    base = [None] * A
                    base[0] = jnp.full((L,), 0, jnp.int32) + (
                        tbase + (T - 1) * D)
                    for j in range(1, A):
                        base[j] = tbase + m[j] * D
                    obase = obase_f + g * (L * D) + laneD
                    for d in range(D):
                        acc = coef[0] * plsc.load_gather(tb, [base[0] + d])
                        for j in range(1, A):
                            acc = acc + coef[j] * plsc.load_gather(
                                tb, [base[j] + d])
                        plsc.store_scatter(ob, [obase + d], acc)
                    return carry2

                return lax.fori_loop(0, NG, gbody, carry)

            lax.fori_loop(0, NF, fbody, 0)
            for fl in range(NF):
                pltpu.sync_copy(
                    ob.at[pl.ds(fl * (BC * D), BC * D)],
                    yt_hbm.at[pl.ds((f0 + fl) * (B * D) + b0c * D, BC * D)])

    return k


def kernel(X, params):
    B, I, A = X.shape
    _, T, D = params.shape
    k = _make_kernel(B, I, A, T, D, BC=128)
    xt = jnp.transpose(X, (1, 2, 0)).reshape(-1)
    yt = k(xt, params.reshape(-1))
    return jnp.transpose(yt.reshape(I, B, D), (1, 0, 2))


# speedup vs baseline: 14.9634x; 1.0379x over previous
import functools

import jax
import jax.numpy as jnp
from jax import lax
from jax.experimental import pallas as pl
from jax.experimental.pallas import tpu as pltpu
from jax.experimental.pallas import tpu_sc as plsc

L = 16
NC = 2
NS = 16
NW = NC * NS

_CES = [(0, 1), (2, 3), (4, 5), (6, 7),
        (0, 2), (1, 3), (4, 6), (5, 7),
        (1, 2), (5, 6),
        (0, 4), (1, 5), (2, 6), (3, 7),
        (2, 4), (3, 5),
        (1, 2), (3, 4), (5, 6)]


def _make_kernel(B, I, A, T, D, BC):
    NF = I // NW
    NG = BC // L
    mesh = plsc.VectorSubcoreMesh(core_axis_name="c", subcore_axis_name="s",
                                  num_cores=NC, num_subcores=NS)

    @functools.partial(
        pl.kernel,
        out_type=jax.ShapeDtypeStruct((I * B * D,), jnp.float32),
        mesh=mesh,
        scratch_types=[
            pltpu.VMEM((NF * T * D,), jnp.float32),
            pltpu.VMEM((NF * A * B,), jnp.float32),
            pltpu.VMEM((NF * BC * D,), jnp.float32),
        ],
        compiler_params=pltpu.CompilerParams(needs_layout_passes=False),
    )
    def k(xt_hbm, pflat_hbm, yt_hbm, tb, xb, ob):
        wid = lax.axis_index("s") * NC + lax.axis_index("c")
        f0 = wid * NF
        pltpu.sync_copy(pflat_hbm.at[pl.ds(f0 * (T * D), NF * T * D)], tb)
        pltpu.sync_copy(xt_hbm.at[pl.ds(f0 * (A * B), NF * A * B)], xb)
        lane = jnp.arange(L, dtype=jnp.int32)
        laneD = lane * D
        pay = [jnp.full((L,), 1 << j, jnp.int32) for j in range(A)]

        for ci in range(B // BC):
            b0c = ci * BC

            @plsc.parallel_loop(0, NF * NG, 1, unroll=2)
            def _group(t):
                fl = t // NG
                g = t - fl * NG
                xfl = fl * (A * B)
                tbase = fl * (T * D)
                b0 = b0c + g * L
                v = [xb[pl.ds(xfl + j * B + b0, L)] for j in range(A)]
                p = list(pay)
                for a, b in _CES:
                    c = v[a] <= v[b]
                    va, vb = v[a], v[b]
                    v[a] = jnp.where(c, va, vb)
                    v[b] = jnp.where(c, vb, va)
                    pa, pb = p[a], p[b]
                    p[a] = jnp.where(c, pa, pb)
                    p[b] = jnp.where(c, pb, pa)
                coef = [v[0]] + [v[j] - v[j - 1] for j in range(1, A)]
                m = [None] * A
                m[A - 1] = p[A - 1]
                for j in range(A - 2, 0, -1):
                    m[j] = m[j + 1] + p[j]
                base = [None] * A
                base[0] = jnp.full((L,), 0, jnp.int32) + (
                    tbase + (T - 1) * D)
                for j in range(1, A):
                    base[j] = tbase + m[j] * D
                obase = t * (L * D) + laneD
                for d in range(D):
                    acc = coef[0] * plsc.load_gather(tb, [base[0] + d])
                    for j in range(1, A):
                        acc = acc + coef[j] * plsc.load_gather(
                            tb, [base[j] + d])
                    plsc.store_scatter(ob, [obase + d], acc)
            for fl in range(NF):
                pltpu.sync_copy(
                    ob.at[pl.ds(fl * (BC * D), BC * D)],
                    yt_hbm.at[pl.ds((f0 + fl) * (B * D) + b0c * D, BC * D)])

    return k


def kernel(X, params):
    B, I, A = X.shape
    _, T, D = params.shape
    k = _make_kernel(B, I, A, T, D, BC=128)
    xt = jnp.transpose(X, (1, 2, 0)).reshape(-1)
    yt = k(xt, params.reshape(-1))
    return jnp.transpose(yt.reshape(I, B, D), (1, 0, 2))


# speedup vs baseline: 19.8984x; 1.3298x over previous
import functools

import jax
import jax.numpy as jnp
import numpy as np
from jax import lax
from jax.experimental import pallas as pl
from jax.experimental.pallas import tpu as pltpu
from jax.experimental.pallas import tpu_sc as plsc

L = 16
NC = 2
NS = 16
NW = NC * NS

_CES = [(0, 1), (2, 3), (4, 5), (6, 7),
        (0, 2), (1, 3), (4, 6), (5, 7),
        (1, 2), (5, 6),
        (0, 4), (1, 5), (2, 6), (3, 7),
        (2, 4), (3, 5),
        (1, 2), (3, 4), (5, 6)]


def _make_kernel(B, I, A, T, D, BC):
    NF = I // NW
    NG = BC // L
    NCH = B // BC
    R = D + 1
    CH = NF * NG * D * L
    mesh = plsc.VectorSubcoreMesh(core_axis_name="c", subcore_axis_name="s",
                                  num_cores=NC, num_subcores=NS)

    @functools.partial(
        pl.kernel,
        out_type=jax.ShapeDtypeStruct((NW * NCH * CH,), jnp.float32),
        mesh=mesh,
        scratch_types=[
            pltpu.VMEM((NF * T * R,), jnp.float32),
            pltpu.VMEM((NF * A * B,), jnp.float32),
            pltpu.VMEM((CH,), jnp.float32),
        ],
        compiler_params=pltpu.CompilerParams(needs_layout_passes=False),
    )
    def k(xt_hbm, pflat_hbm, y_hbm, tb, xb, ob):
        wid = lax.axis_index("s") * NC + lax.axis_index("c")
        f0 = wid * NF
        pltpu.sync_copy(pflat_hbm.at[pl.ds(f0 * (T * R), NF * T * R)], tb)
        pltpu.sync_copy(xt_hbm.at[pl.ds(f0 * (A * B), NF * A * B)], xb)
        pay = [jnp.full((L,), 1 << j, jnp.int32) for j in range(A)]
        j0_off = (255 ^ 15) * R

        def cbody(ci, carry):
            b0c = ci * BC

            @plsc.parallel_loop(0, NF * NG, 1, unroll=2)
            def _group(t):
                fl = t // NG
                g = t - fl * NG
                xfl = fl * (A * B)
                tbase = fl * (T * R)
                b0 = b0c + g * L
                v = [xb[pl.ds(xfl + j * B + b0, L)] for j in range(A)]
                p = list(pay)
                for a, b in _CES:
                    c = v[a] <= v[b]
                    va, vb = v[a], v[b]
                    v[a] = jnp.where(c, va, vb)
                    v[b] = jnp.where(c, vb, va)
                    pa, pb = p[a], p[b]
                    p[a] = jnp.where(c, pa, pb)
                    p[b] = jnp.where(c, pb, pa)
                coef = [v[0]] + [v[j] - v[j - 1] for j in range(1, A)]
                m = [None] * A
                m[A - 1] = p[A - 1]
                for j in range(A - 2, 0, -1):
                    m[j] = m[j + 1] + p[j]
                base = [None] * A
                base[0] = jnp.full((L,), 0, jnp.int32) + (tbase + j0_off)
                for j in range(1, A):
                    s = m[j] ^ (m[j] >> 4)
                    base[j] = tbase + s * R
                od = t * (D * L)
                for d in range(D):
                    acc = coef[0] * plsc.load_gather(tb, [base[0] + d])
                    for j in range(1, A):
                        acc = acc + coef[j] * plsc.load_gather(
                            tb, [base[j] + d])
                    ob[pl.ds(od + d * L, L)] = acc

            pltpu.sync_copy(
                ob, y_hbm.at[pl.ds((wid * NCH + ci) * CH, CH)])
            return carry

        lax.fori_loop(0, NCH, cbody, 0)

    return k


def kernel(X, params):
    B, I, A = X.shape
    _, T, D = params.shape
    BC = 128
    k = _make_kernel(B, I, A, T, D, BC=BC)
    xt = jnp.transpose(X, (1, 2, 0)).reshape(-1)
    sv = np.arange(T) ^ (np.arange(T) >> 4)
    sinv = np.argsort(sv)
    ps = jnp.pad(params[:, sinv, :], ((0, 0), (0, 0), (0, 1)))
    y_raw = k(xt, ps.reshape(-1))
    NF, NG, NCH = I // NW, BC // 16, B // BC
    y6 = y_raw.reshape(NW, NCH, NF, NG, D, 16)
    return jnp.transpose(y6, (1, 3, 5, 0, 2, 4)).reshape(B, I, D)


# speedup vs baseline: 34.5609x; 1.7369x over previous
import functools

import jax
import jax.numpy as jnp
from jax import lax
from jax.experimental import pallas as pl
from jax.experimental.pallas import tpu as pltpu
from jax.experimental.pallas import tpu_sc as plsc

L = 16
NC = 2
NS = 16
NW = NC * NS

_CES = [(0, 1), (2, 3), (4, 5), (6, 7),
        (0, 2), (1, 3), (4, 6), (5, 7),
        (1, 2), (5, 6),
        (0, 4), (1, 5), (2, 6), (3, 7),
        (2, 4), (3, 5),
        (1, 2), (3, 4), (5, 6)]


def _make_kernel(B, I, A, T, D, BC):
    NF = I // NW
    NG = BC // L
    NCH = B // BC
    R = T + 1
    TS = D + 1
    XS = NF * A + 1
    OS = NF * D + 1
    NTC = 2
    mesh = plsc.VectorSubcoreMesh(core_axis_name="c", subcore_axis_name="s",
                                  num_cores=NC, num_subcores=NS)

    @functools.partial(
        pl.kernel,
        out_type=jax.ShapeDtypeStruct((B, I * D), jnp.float32),
        mesh=mesh,
        scratch_types=[
            pltpu.VMEM((NTC * T * D,), jnp.float32),
            pltpu.VMEM((NF * T * TS,), jnp.float32),
            pltpu.VMEM((BC, XS), jnp.float32),
            pltpu.VMEM((BC, OS), jnp.float32),
        ],
        compiler_params=pltpu.CompilerParams(
            needs_layout_passes=False, use_tc_tiling_on_sc=False),
    )
    def k(x2_hbm, pflat_hbm, y2_hbm, traw, tb, xb, ob):
        wid = lax.axis_index("s") * NC + lax.axis_index("c")
        f0 = wid * NF
        lane = jnp.arange(L, dtype=jnp.int32)
        pay = [jnp.full((L,), 1 << j, jnp.int32) for j in range(A)]
        j0_off = (255 ^ 15) * TS

        for tc in range(NF // NTC):
            pltpu.sync_copy(
                pflat_hbm.at[pl.ds((f0 + tc * NTC) * (T * D), NTC * T * D)],
                traw)

            @plsc.parallel_loop(0, NTC * T, 1, unroll=4)
            def _srow(r):
                m = r & (T - 1)
                fc = r >> 8
                s = m ^ (m >> 4)
                dst = ((tc * NTC + fc) * T + s) * TS
                tb[pl.ds(dst, D)] = traw[pl.ds(r * D, D)]

        def cbody(ci, carry):
            b0c = ci * BC
            pltpu.sync_copy(
                x2_hbm.at[pl.ds(b0c, BC), pl.ds(f0 * A, NF * A)],
                xb.at[:, pl.ds(0, NF * A)])

            @plsc.parallel_loop(0, NF * NG, 1, unroll=2)
            def _group(t):
                fl = t // NG
                g = t - fl * NG
                tbase = fl * (T * TS)
                bloc = g * L + lane
                xrow = bloc * XS + fl * A
                v = [plsc.load_gather(xb, [bloc, jnp.full(
                        (L,), fl * A + j, jnp.int32)]) for j in range(A)]
                p = list(pay)
                for a, b in _CES:
                    c = v[a] <= v[b]
                    va, vb = v[a], v[b]
                    v[a] = jnp.where(c, va, vb)
                    v[b] = jnp.where(c, vb, va)
                    pa, pb = p[a], p[b]
                    p[a] = jnp.where(c, pa, pb)
                    p[b] = jnp.where(c, pb, pa)
                coef = [v[0]] + [v[j] - v[j - 1] for j in range(1, A)]
                m = [None] * A
                m[A - 1] = p[A - 1]
                for j in range(A - 2, 0, -1):
                    m[j] = m[j + 1] + p[j]
                base = [None] * A
                base[0] = jnp.full((L,), 0, jnp.int32) + (tbase + j0_off)
                for j in range(1, A):
                    s = m[j] ^ (m[j] >> 4)
                    base[j] = tbase + s * TS
                for d in range(D):
                    acc = coef[0] * plsc.load_gather(tb, [base[0] + d])
                    for j in range(1, A):
                        acc = acc + coef[j] * plsc.load_gather(
                            tb, [base[j] + d])
                    plsc.store_scatter(
                        ob, [bloc, jnp.full((L,), fl * D + d, jnp.int32)],
                        acc)

            pltpu.sync_copy(
                ob.at[:, pl.ds(0, NF * D)],
                y2_hbm.at[pl.ds(b0c, BC), pl.ds(f0 * D, NF * D)])
            return carry

        lax.fori_loop(0, NCH, cbody, 0)

    return k


def kernel(X, params):
    B, I, A = X.shape
    _, T, D = params.shape
    k = _make_kernel(B, I, A, T, D, BC=128)
    y2 = k(X.reshape(B, I * A), params.reshape(-1))
    return y2.reshape(B, I, D)
